# baseline (speedup 1.0000x reference)
import functools

import jax
import jax.numpy as jnp
from jax import lax
from jax.experimental import pallas as pl
from jax.experimental.pallas import tpu as pltpu
from jax.experimental.pallas import tpu_sc as plsc

N = 16384
D = 128
NSUBJ = 64
PD = 16
_PACK = 128 // PD
NG = N // _PACK
_BT = 2048
_PB = _BT // _PACK

_NC = 2
_NS = 16
_NW = _NC * _NS
_BPW = N // _NW
_L = 16

_sc_mesh = plsc.VectorSubcoreMesh(core_axis_name="c", subcore_axis_name="s")


@functools.partial(
    pl.kernel,
    out_type=jax.ShapeDtypeStruct((NG, 128), jnp.float32),
    mesh=_sc_mesh,
    scratch_types=[
        pltpu.VMEM((NSUBJ, 128), jnp.float32),
        pltpu.VMEM((_BPW,), jnp.int32),
        pltpu.VMEM((256, 32), jnp.float32),
    ],
    compiler_params=pltpu.CompilerParams(
        use_tc_tiling_on_sc=False, needs_layout_passes=False
    ),
)
def _sc_gather(tab_hbm, sid_hbm, out_hbm, tab_v, idx_v, out_v):
    wid = lax.axis_index("s") * _NC + lax.axis_index("c")
    iblk = wid // 4
    q = wid % 4
    pltpu.sync_copy(tab_hbm, tab_v)
    pltpu.sync_copy(sid_hbm.at[pl.ds(wid * _BPW, _BPW)], idx_v)
    lanes = lax.iota(jnp.int32, _L)

    def body(g, carry):
        sidv = idx_v[pl.ds(g * _L, _L)]
        rows = lanes + (g % 16) * _L
        cbase = (g // 16) * _L
        for j in range(6):
            colj = jnp.full((_L,), j, jnp.int32) + cbase
            vals = plsc.load_gather(tab_v, [sidv, jnp.full((_L,), j, jnp.int32)])
            plsc.store_scatter(out_v, [rows, colj], vals)
        return carry

    lax.fori_loop(0, _BPW // _L, body, 0)
    pltpu.sync_copy(out_v, out_hbm.at[pl.ds(iblk * 256, 256), pl.ds(q * 32, 32)])


def _selectors():
    r2 = lax.broadcasted_iota(jnp.int32, (2, PD), 0)
    c2 = lax.broadcasted_iota(jnp.int32, (2, PD), 1)
    s_sel = (c2 // 2 == r2).astype(jnp.float32)
    c1 = lax.broadcasted_iota(jnp.int32, (1, PD), 1)
    c_sel = (c1 // 2 == 2).astype(jnp.float32)
    return s_sel, c_sel


def _tc_expand(x0, x1, x2, x3, x4, x5, x6, x7, w_ref, b_ref, y_ref):
    s_sel, c_sel = _selectors()
    w2 = jnp.dot(w_ref[...], s_sel, preferred_element_type=jnp.float32)
    w2b = w2.astype(jnp.bfloat16)
    cvec = jnp.dot(b_ref[...], s_sel, preferred_element_type=jnp.float32) + c_sel
    pieces = []
    for xp in (x0, x1, x2, x3, x4, x5, x6, x7):
        z = jnp.dot(
            xp[...].astype(jnp.bfloat16), w2b, preferred_element_type=jnp.float32
        )
        pieces.append(z + cvec)
    y_ref[...] = jnp.concatenate(pieces, axis=1)


def _tc_combine(y_ref, g_ref, o_ref):
    rj = lax.broadcasted_iota(jnp.int32, (128, PD), 0)
    ck = lax.broadcasted_iota(jnp.int32, (128, PD), 1)
    rbig = ((rj // PD == ck // 2) & (rj % PD < 6) & (rj % 2 == ck % 2)).astype(
        jnp.float32
    )
    t = y_ref[...] * g_ref[...]
    o_ref[...] = jnp.dot(t, rbig, preferred_element_type=jnp.float32)


def kernel(x, subjectID, W_net, b_net, W_cal, b_cal):
    tab = jnp.concatenate(
        [W_cal.reshape(NSUBJ, 4), b_cal, jnp.zeros((NSUBJ, 122), jnp.float32)],
        axis=1,
    )
    g = _sc_gather(tab, subjectID.astype(jnp.int32))
    ybig = pl.pallas_call(
        _tc_expand,
        grid=(N // _BT,),
        in_specs=[
            pl.BlockSpec((_PB, D), functools.partial(lambda p, i: (8 * i + p, 0), p))
            for p in range(_PACK)
        ]
        + [
            pl.BlockSpec((D, 2), lambda i: (0, 0)),
            pl.BlockSpec((1, 2), lambda i: (0, 0)),
        ],
        out_specs=pl.BlockSpec((_PB, 128), lambda i: (i, 0)),
        out_shape=jax.ShapeDtypeStruct((NG, 128), jnp.float32),
    )(x, x, x, x, x, x, x, x, W_net, b_net.reshape(1, 2))
    op = pl.pallas_call(
        _tc_combine,
        grid=(N // _BT,),
        in_specs=[
            pl.BlockSpec((_PB, 128), lambda i: (i, 0)),
            pl.BlockSpec((_PB, 128), lambda i: (i, 0)),
        ],
        out_specs=pl.BlockSpec((_PB, PD), lambda i: (i, 0)),
        out_shape=jax.ShapeDtypeStruct((NG, PD), jnp.float32),
    )(ybig, g)
    out = op.reshape(8, _PB, _PACK, 2).transpose(0, 2, 1, 3).reshape(N, 2)
    return out


# speedup vs baseline: 1.0035x; 1.0035x over previous
import functools

import jax
import jax.numpy as jnp
from jax import lax
from jax.experimental import pallas as pl
from jax.experimental.pallas import tpu as pltpu
from jax.experimental.pallas import tpu_sc as plsc

N = 16384
D = 128
NSUBJ = 64
PD = 16
_PACK = 128 // PD
NG = N // _PACK
_BT = 2048
_PB = _BT // _PACK

_NC = 2
_NS = 16
_NW = _NC * _NS
_BPW = N // _NW
_L = 16

_sc_mesh = plsc.VectorSubcoreMesh(core_axis_name="c", subcore_axis_name="s")


@functools.partial(
    pl.kernel,
    out_type=jax.ShapeDtypeStruct((NG, 128), jnp.float32),
    mesh=_sc_mesh,
    scratch_types=[
        pltpu.VMEM((NSUBJ, 128), jnp.float32),
        pltpu.VMEM((_BPW,), jnp.int32),
        pltpu.VMEM((256, 32), jnp.float32),
    ],
    compiler_params=pltpu.CompilerParams(
        use_tc_tiling_on_sc=False, needs_layout_passes=False
    ),
)
def _sc_gather(tab_hbm, sid_hbm, out_hbm, tab_v, idx_v, out_v):
    wid = lax.axis_index("s") * _NC + lax.axis_index("c")
    iblk = wid // 4
    q = wid % 4
    pltpu.sync_copy(tab_hbm, tab_v)
    pltpu.sync_copy(sid_hbm.at[pl.ds(wid * _BPW, _BPW)], idx_v)
    lanes = lax.iota(jnp.int32, _L)

    def body(g, carry):
        sidv = idx_v[pl.ds(g * _L, _L)]
        rows = lanes + (g % 16) * _L
        cbase = (g // 16) * _L
        for j in range(6):
            colj = jnp.full((_L,), j, jnp.int32) + cbase
            vals = plsc.load_gather(tab_v, [sidv, jnp.full((_L,), j, jnp.int32)])
            plsc.store_scatter(out_v, [rows, colj], vals)
        return carry

    lax.fori_loop(0, _BPW // _L, body, 0)
    pltpu.sync_copy(out_v, out_hbm.at[pl.ds(iblk * 256, 256), pl.ds(q * 32, 32)])


def _selectors():
    r2 = lax.broadcasted_iota(jnp.int32, (2, PD), 0)
    c2 = lax.broadcasted_iota(jnp.int32, (2, PD), 1)
    s_sel = (c2 // 2 == r2).astype(jnp.float32)
    c1 = lax.broadcasted_iota(jnp.int32, (1, PD), 1)
    c_sel = (c1 // 2 == 2).astype(jnp.float32)
    return s_sel, c_sel


def _tc_expand(x_ref, w_ref, b_ref, y_ref):
    s_sel, c_sel = _selectors()
    w2 = jnp.dot(w_ref[...], s_sel, preferred_element_type=jnp.float32)
    w2b = w2.astype(jnp.bfloat16)
    cvec = jnp.dot(b_ref[...], s_sel, preferred_element_type=jnp.float32) + c_sel
    xb = x_ref[...].astype(jnp.bfloat16)
    pieces = []
    for p in range(_PACK):
        z = jnp.dot(
            xb[_PB * p : _PB * (p + 1), :], w2b, preferred_element_type=jnp.float32
        )
        pieces.append(z + cvec)
    y_ref[...] = jnp.concatenate(pieces, axis=1)


def _tc_combine(y_ref, g_ref, o_ref):
    rj = lax.broadcasted_iota(jnp.int32, (128, PD), 0)
    ck = lax.broadcasted_iota(jnp.int32, (128, PD), 1)
    rbig = ((rj // PD == ck // 2) & (rj % PD < 6) & (rj % 2 == ck % 2)).astype(
        jnp.float32
    )
    t = y_ref[...] * g_ref[...]
    o_ref[...] = jnp.dot(t, rbig, preferred_element_type=jnp.float32)


def kernel(x, subjectID, W_net, b_net, W_cal, b_cal):
    tab = jnp.concatenate(
        [W_cal.reshape(NSUBJ, 4), b_cal, jnp.zeros((NSUBJ, 122), jnp.float32)],
        axis=1,
    )
    g = _sc_gather(tab, subjectID.astype(jnp.int32))
    ybig = pl.pallas_call(
        _tc_expand,
        grid=(N // _BT,),
        in_specs=[
            pl.BlockSpec((_BT, D), lambda i: (i, 0)),
            pl.BlockSpec((D, 2), lambda i: (0, 0)),
            pl.BlockSpec((1, 2), lambda i: (0, 0)),
        ],
        out_specs=pl.BlockSpec((_PB, 128), lambda i: (i, 0)),
        out_shape=jax.ShapeDtypeStruct((NG, 128), jnp.float32),
    )(x, W_net, b_net.reshape(1, 2))
    op = pl.pallas_call(
        _tc_combine,
        grid=(N // _BT,),
        in_specs=[
            pl.BlockSpec((_PB, 128), lambda i: (i, 0)),
            pl.BlockSpec((_PB, 128), lambda i: (i, 0)),
        ],
        out_specs=pl.BlockSpec((_PB, PD), lambda i: (i, 0)),
        out_shape=jax.ShapeDtypeStruct((NG, PD), jnp.float32),
    )(ybig, g)
    out = op.reshape(8, _PB, _PACK, 2).transpose(0, 2, 1, 3).reshape(N, 2)
    return out


# speedup vs baseline: 1.1707x; 1.1666x over previous
import functools

import jax
import jax.numpy as jnp
from jax import lax
from jax.experimental import pallas as pl
from jax.experimental.pallas import tpu as pltpu
from jax.experimental.pallas import tpu_sc as plsc

N = 16384
D = 128
NSUBJ = 64
PD = 16
_PACK = 128 // PD
NG = N // _PACK
_BT = 2048
_PB = _BT // _PACK

_NC = 2
_NS = 16
_NW = _NC * _NS
_BPW = N // _NW
_L = 16

_sc_mesh = plsc.VectorSubcoreMesh(core_axis_name="c", subcore_axis_name="s")


@functools.partial(
    pl.kernel,
    out_type=jax.ShapeDtypeStruct((NG, 128), jnp.float32),
    mesh=_sc_mesh,
    scratch_types=[
        pltpu.VMEM((NSUBJ, 128), jnp.float32),
        pltpu.VMEM((_BPW,), jnp.int32),
        pltpu.VMEM((256, 32), jnp.float32),
    ],
    compiler_params=pltpu.CompilerParams(
        use_tc_tiling_on_sc=False, needs_layout_passes=False
    ),
)
def _sc_gather(tab_hbm, sid_hbm, out_hbm, tab_v, idx_v, out_v):
    wid = lax.axis_index("s") * _NC + lax.axis_index("c")
    iblk = wid // 4
    q = wid % 4
    pltpu.sync_copy(tab_hbm, tab_v)
    pltpu.sync_copy(sid_hbm.at[pl.ds(wid * _BPW, _BPW)], idx_v)
    lanes = lax.iota(jnp.int32, _L)

    def body(g, carry):
        sidv = idx_v[pl.ds(g * _L, _L)]
        rows = lanes + (g % 16) * _L
        cbase = (g // 16) * _L
        for j in range(6):
            colj = jnp.full((_L,), j, jnp.int32) + cbase
            vals = plsc.load_gather(tab_v, [sidv, jnp.full((_L,), j, jnp.int32)])
            plsc.store_scatter(out_v, [rows, colj], vals)
        return carry

    lax.fori_loop(0, _BPW // _L, body, 0)
    pltpu.sync_copy(out_v, out_hbm.at[pl.ds(iblk * 256, 256), pl.ds(q * 32, 32)])


def _selectors():
    r2 = lax.broadcasted_iota(jnp.int32, (2, PD), 0)
    c2 = lax.broadcasted_iota(jnp.int32, (2, PD), 1)
    s_sel = (c2 // 2 == r2).astype(jnp.float32)
    c1 = lax.broadcasted_iota(jnp.int32, (1, PD), 1)
    c_sel = (c1 // 2 == 2).astype(jnp.float32)
    return s_sel, c_sel


def _tc_expand(x_ref, w_ref, b_ref, y_ref):
    s_sel, c_sel = _selectors()
    w2 = jnp.dot(w_ref[...], s_sel, preferred_element_type=jnp.float32)
    w2b = w2.astype(jnp.bfloat16)
    cvec = jnp.dot(b_ref[...], s_sel, preferred_element_type=jnp.float32) + c_sel
    xb = x_ref[...].astype(jnp.bfloat16)
    halves = []
    for h in range(2):
        pieces = []
        for p in range(_PACK):
            o = _BT * h + _PB * p
            z = jnp.dot(xb[o : o + _PB, :], w2b, preferred_element_type=jnp.float32)
            pieces.append(z + cvec)
        halves.append(jnp.concatenate(pieces, axis=1))
    y_ref[...] = jnp.concatenate(halves, axis=0)


def _tc_combine(y_ref, g_ref, o_ref):
    rj = lax.broadcasted_iota(jnp.int32, (128, PD), 0)
    ck = lax.broadcasted_iota(jnp.int32, (128, PD), 1)
    rbig = ((rj // PD == ck // 2) & (rj % PD < 6) & (rj % 2 == ck % 2)).astype(
        jnp.float32
    )
    t = y_ref[...] * g_ref[...]
    o_ref[...] = jnp.dot(t, rbig, preferred_element_type=jnp.float32)


def kernel(x, subjectID, W_net, b_net, W_cal, b_cal):
    tab = jnp.concatenate(
        [W_cal.reshape(NSUBJ, 4), b_cal, jnp.zeros((NSUBJ, 122), jnp.float32)],
        axis=1,
    )
    g = _sc_gather(tab, subjectID.astype(jnp.int32))
    ybig = pl.pallas_call(
        _tc_expand,
        grid=(N // (2 * _BT),),
        in_specs=[
            pl.BlockSpec((2 * _BT, D), lambda i: (i, 0)),
            pl.BlockSpec((D, 2), lambda i: (0, 0)),
            pl.BlockSpec((1, 2), lambda i: (0, 0)),
        ],
        out_specs=pl.BlockSpec((2 * _PB, 128), lambda i: (i, 0)),
        out_shape=jax.ShapeDtypeStruct((NG, 128), jnp.float32),
    )(x, W_net, b_net.reshape(1, 2))
    op = pl.pallas_call(
        _tc_combine,
        grid=(2,),
        in_specs=[
            pl.BlockSpec((NG // 2, 128), lambda i: (i, 0)),
            pl.BlockSpec((NG // 2, 128), lambda i: (i, 0)),
        ],
        out_specs=pl.BlockSpec((NG // 2, PD), lambda i: (i, 0)),
        out_shape=jax.ShapeDtypeStruct((NG, PD), jnp.float32),
    )(ybig, g)
    out = op.reshape(8, _PB, _PACK, 2).transpose(0, 2, 1, 3).reshape(N, 2)
    return out
